# baseline (speedup 1.0000x reference)
import jax
import jax.numpy as jnp
from jax.experimental import pallas as pl
from jax.experimental.pallas import tpu as pltpu

_N, _D, _P, _E = 32768, 1024, 256, 64
_BC = 2048
_NB = _N // _BC
_S = 4


def _start(x_hbm, xbuf, sem, blk):
    pltpu.make_async_copy(
        x_hbm.at[pl.ds(blk * _BC, _BC), :],
        xbuf.at[blk % _S],
        sem.at[blk % _S],
    ).start()


def _router_kernel(x_hbm, w_ref, b_ref, sim_ref, t_ref, dummy_ref, out_ref,
                   xbuf, sem):
    del dummy_ref
    i = pl.program_id(0)

    @pl.when(i == 0)
    def _prologue():
        for s in range(_S - 1):
            _start(x_hbm, xbuf, sem, s)

    j = i + _S - 1

    @pl.when(j < _NB)
    def _prefetch():
        _start(x_hbm, xbuf, sem, j)

    pltpu.make_async_copy(
        x_hbm.at[pl.ds(i * _BC, _BC), :],
        xbuf.at[i % _S],
        sem.at[i % _S],
    ).wait()

    x = xbuf[i % _S]
    w = w_ref[...]
    proj = jax.lax.dot_general(
        x, w, (((1,), (1,)), ((), ())), preferred_element_type=jnp.float32
    )
    proj = proj + b_ref[...]
    norm = jnp.sqrt(jnp.sum(proj * proj, axis=1, keepdims=True))
    norm = jnp.maximum(norm, 1e-12)
    sim = sim_ref[...]
    sim_n = sim / jnp.maximum(
        jnp.sqrt(jnp.sum(sim * sim, axis=0, keepdims=True)), 1e-12
    )
    raw = jax.lax.dot_general(
        proj, sim_n, (((1,), (0,)), ((), ())), preferred_element_type=jnp.float32
    )
    clamp_max = jnp.log(jnp.float32(1.0 / 0.01))
    scale = jnp.exp(jnp.minimum(t_ref[0, 0], clamp_max))
    logits = raw * (scale / norm)
    m = jnp.max(logits, axis=1, keepdims=True)
    e = jnp.exp(logits - m)
    out_ref[...] = e / jnp.sum(e, axis=1, keepdims=True)


@jax.jit
def kernel(x, W, b, sim, temperature):
    b2 = b.reshape(1, _P)
    t2 = temperature.reshape(1, 1)
    grid = (_NB,)
    return pl.pallas_call(
        _router_kernel,
        grid=grid,
        in_specs=[
            pl.BlockSpec(memory_space=pltpu.MemorySpace.HBM),
            pl.BlockSpec((_P, _D), lambda i: (0, 0)),
            pl.BlockSpec((1, _P), lambda i: (0, 0)),
            pl.BlockSpec((_P, _E), lambda i: (0, 0)),
            pl.BlockSpec((1, 1), lambda i: (0, 0)),
            pl.BlockSpec(memory_space=pltpu.MemorySpace.HBM),
        ],
        out_specs=pl.BlockSpec((_BC, _E), lambda i: (i, 0)),
        out_shape=jax.ShapeDtypeStruct((_N, _E), jnp.float32),
        scratch_shapes=[
            pltpu.VMEM((_S, _BC, _D), jnp.float32),
            pltpu.SemaphoreType.DMA((_S,)),
        ],
        input_output_aliases={5: 0},
        compiler_params=pltpu.CompilerParams(
            dimension_semantics=("arbitrary",),
        ),
    )(x, W, b2, sim, t2, jnp.zeros((_N, _E), jnp.float32))


# speedup vs baseline: 1.0933x; 1.0933x over previous
import jax
import jax.numpy as jnp
from jax.experimental import pallas as pl
from jax.experimental.pallas import tpu as pltpu

_N, _D, _P, _E = 32768, 1024, 256, 64
_BC = 2048
_NB = _N // _BC
_S = 4


def _start(x_hbm, xbuf, sem, blk):
    pltpu.make_async_copy(
        x_hbm.at[pl.ds(blk * _BC, _BC), :],
        xbuf.at[blk % _S],
        sem.at[blk % _S],
    ).start()


def _router_kernel(x_hbm, w_ref, b_ref, sim_ref, t_ref, out_ref, xbuf, sem):
    i = pl.program_id(0)

    @pl.when(i == 0)
    def _prologue():
        for s in range(_S - 1):
            _start(x_hbm, xbuf, sem, s)

    j = i + _S - 1

    @pl.when(j < _NB)
    def _prefetch():
        _start(x_hbm, xbuf, sem, j)

    pltpu.make_async_copy(
        x_hbm.at[pl.ds(i * _BC, _BC), :],
        xbuf.at[i % _S],
        sem.at[i % _S],
    ).wait()

    x = xbuf[i % _S]
    w = w_ref[...]
    proj = jax.lax.dot_general(
        x, w, (((1,), (1,)), ((), ())), preferred_element_type=jnp.float32
    )
    proj = proj + b_ref[...]
    norm = jnp.sqrt(jnp.sum(proj * proj, axis=1, keepdims=True))
    norm = jnp.maximum(norm, 1e-12)
    sim = sim_ref[...]
    sim_n = sim / jnp.maximum(
        jnp.sqrt(jnp.sum(sim * sim, axis=0, keepdims=True)), 1e-12
    )
    raw = jax.lax.dot_general(
        proj, sim_n, (((1,), (0,)), ((), ())), preferred_element_type=jnp.float32
    )
    clamp_max = jnp.log(jnp.float32(1.0 / 0.01))
    scale = jnp.exp(jnp.minimum(t_ref[0, 0], clamp_max))
    logits = raw * (scale / norm)
    m = jnp.max(logits, axis=1, keepdims=True)
    e = jnp.exp(logits - m)
    out_ref[...] = e / jnp.sum(e, axis=1, keepdims=True)


@jax.jit
def kernel(x, W, b, sim, temperature):
    b2 = b.reshape(1, _P)
    t2 = temperature.reshape(1, 1)
    grid = (_NB,)
    return pl.pallas_call(
        _router_kernel,
        grid=grid,
        in_specs=[
            pl.BlockSpec(memory_space=pltpu.MemorySpace.HBM),
            pl.BlockSpec((_P, _D), lambda i: (0, 0)),
            pl.BlockSpec((1, _P), lambda i: (0, 0)),
            pl.BlockSpec((_P, _E), lambda i: (0, 0)),
            pl.BlockSpec((1, 1), lambda i: (0, 0)),
        ],
        out_specs=pl.BlockSpec((_BC, _E), lambda i: (i, 0)),
        out_shape=jax.ShapeDtypeStruct((_N, _E), jnp.float32),
        scratch_shapes=[
            pltpu.VMEM((_S, _BC, _D), jnp.float32),
            pltpu.SemaphoreType.DMA((_S,)),
        ],
        compiler_params=pltpu.CompilerParams(
            dimension_semantics=("arbitrary",),
        ),
    )(x, W, b2, sim, t2)
